# baseline (speedup 1.0000x reference)
import functools
import math

import jax
import jax.numpy as jnp
from jax.experimental import pallas as pl
from jax.experimental.pallas import tpu as pltpu

_EPS = 1e-5
_BF = jnp.bfloat16
_F32 = jnp.float32


def _row_cat(a):
    nb, H, W, C = a.shape
    zrow = jnp.zeros((nb, 1, W, C), a.dtype)
    up = jnp.concatenate([zrow, a[:, :-1]], axis=1)
    dn = jnp.concatenate([a[:, 1:], zrow], axis=1)
    return jnp.concatenate([up, a, dn], axis=3)


def _bn_affine(s, q, count, gamma, beta):
    mu = s / count
    var = q / count - mu * mu
    scale = gamma * jax.lax.rsqrt(var + _EPS)
    return scale, beta - mu * scale


def _conv_block(inp_bf, wc):
    nb, H, W, C = inp_bf.shape
    cat = _row_cat(inp_bf)
    z = jnp.dot(cat.reshape(nb * H * W, 3 * C), wc,
                preferred_element_type=_F32).reshape(nb, H, W, 256)
    zcol = jnp.zeros((nb, H, 1, C), _F32)
    t0 = jnp.concatenate([zcol, z[:, :, :-1, 0:64]], axis=2)
    t2 = jnp.concatenate([z[:, :, 1:, 192:256], zcol], axis=2)
    y = z[:, :, :, 64:128] + t0 + t2
    return z, y


def _fused_kernel(x_ref, w1_ref, ws_ref, g1_ref, be1_ref, w2_ref,
                  g2_ref, be2_ref, gs_ref, bes_ref, o_ref,
                  y1_s, ys_s, st1_s, st2_s, *, nb, count):
    p = pl.program_id(0)
    t = pl.program_id(1)
    H = W = 16
    C = 64
    S = H * W
    z64 = jnp.zeros((C, C), _F32)

    @pl.when(p == 0)
    def _conv1_phase():
        w1c = jnp.concatenate(
            [jnp.concatenate([w1_ref[ky, 0], w1_ref[ky, 1],
                              ws_ref[...] if ky == 1 else z64,
                              w1_ref[ky, 2]], axis=1) for ky in range(3)],
            axis=0).astype(_BF)
        xb = jnp.swapaxes(x_ref[...].astype(_BF), 1, 2).reshape(nb, H, W, C)
        z, y1 = _conv_block(xb, w1c)
        ys = z[:, :, :, 128:192]
        y1_s[pl.ds(t * nb, nb)] = y1.astype(_BF)
        ys_s[pl.ds(t * nb, nb)] = ys.astype(_BF)
        y1r = y1.reshape(nb * S, C)
        ysr = ys.reshape(nb * S, C)
        st = jnp.concatenate(
            [jnp.concatenate([jnp.sum(y1r, axis=0, keepdims=True),
                              jnp.sum(ysr, axis=0, keepdims=True)], axis=1),
             jnp.concatenate([jnp.sum(y1r * y1r, axis=0, keepdims=True),
                              jnp.sum(ysr * ysr, axis=0, keepdims=True)],
                             axis=1)], axis=0)

        @pl.when(t == 0)
        def _():
            st1_s[0:2, :] = st

        @pl.when(t != 0)
        def _():
            st1_s[0:2, :] = st1_s[0:2, :] + st

    @pl.when(p == 1)
    def _conv2_phase():
        w2c = jnp.concatenate(
            [jnp.concatenate([w2_ref[ky, 0], w2_ref[ky, 1], z64,
                              w2_ref[ky, 2]], axis=1) for ky in range(3)],
            axis=0).astype(_BF)
        sc, sh = _bn_affine(st1_s[0:1, 0:C], st1_s[1:2, 0:C], count,
                            g1_ref[...], be1_ref[...])
        scb = sc.astype(_BF).reshape(1, 1, 1, C)
        shb = sh.astype(_BF).reshape(1, 1, 1, C)
        y1 = y1_s[pl.ds(t * nb, nb)]
        a = jnp.maximum(y1 * scb + shb, jnp.bfloat16(0.0))
        _, y2 = _conv_block(a, w2c)
        y1_s[pl.ds(t * nb, nb)] = y2.astype(_BF)
        yr = y2.reshape(nb * S, C)
        st = jnp.concatenate(
            [jnp.sum(yr, axis=0, keepdims=True),
             jnp.sum(yr * yr, axis=0, keepdims=True)], axis=0)

        @pl.when(t == 0)
        def _():
            st2_s[0:2, :] = st

        @pl.when(t != 0)
        def _():
            st2_s[0:2, :] = st2_s[0:2, :] + st

    @pl.when(p == 2)
    def _final_phase():
        sc2, sh2 = _bn_affine(st2_s[0:1, :], st2_s[1:2, :], count,
                              g2_ref[...], be2_ref[...])
        scs, shs = _bn_affine(st1_s[0:1, C:2 * C], st1_s[1:2, C:2 * C], count,
                              gs_ref[...], bes_ref[...])
        vt = jnp.swapaxes(jnp.concatenate([sc2, sh2, scs, shs], axis=0),
                          0, 1).astype(_BF)
        y2t = jnp.swapaxes(y1_s[pl.ds(t * nb, nb)].reshape(nb, S, C), 1, 2)
        yst = jnp.swapaxes(ys_s[pl.ds(t * nb, nb)].reshape(nb, S, C), 1, 2)
        out = (y2t * vt[:, 0:1] + vt[:, 1:2]
               + yst * vt[:, 2:3] + vt[:, 3:4])
        o_ref[...] = jnp.maximum(out, jnp.bfloat16(0.0)).astype(_F32)


@jax.jit
def _forward(x, w1, g1, be1, w2, g2, be2, ws, gs, bes):
    N, Cin, H, W = x.shape
    Cout = w1.shape[-1]
    S = H * W
    x2 = x.reshape(N, Cin, S)
    count = float(N * S)

    nb = math.gcd(N, 32)
    T = N // nb
    cp = pltpu.CompilerParams(
        dimension_semantics=("arbitrary", "arbitrary"),
        vmem_limit_bytes=100 * 1024 * 1024)

    def full(a):
        return pl.BlockSpec(a.shape, lambda p, t: (0,) * a.ndim)

    o = pl.pallas_call(
        functools.partial(_fused_kernel, nb=nb, count=count),
        grid=(3, T),
        in_specs=[pl.BlockSpec((nb, Cin, S), lambda p, t: (t, 0, 0)),
                  full(w1), full(ws), full(g1), full(be1),
                  full(w2), full(g2), full(be2), full(gs), full(bes)],
        out_specs=pl.BlockSpec((nb, Cout, S),
                               lambda p, t: (jnp.where(p == 2, t, 0), 0, 0)),
        out_shape=jax.ShapeDtypeStruct((N, Cout, S), _F32),
        scratch_shapes=[pltpu.VMEM((N, H, W, Cout), _BF),
                        pltpu.VMEM((N, H, W, Cout), _BF),
                        pltpu.VMEM((8, 2 * Cout), _F32),
                        pltpu.VMEM((8, Cout), _F32)],
        compiler_params=cp)(x2, w1, ws, g1, be1, w2, g2, be2, gs, bes)

    return o.reshape(N, Cout, H, W)


def kernel(x, w1, b1, g1, be1, w2, b2, g2, be2, ws, bs, gs, bes):
    return _forward(x, w1, g1, be1, w2, g2, be2, ws, gs, bes)


# speedup vs baseline: 1.0181x; 1.0181x over previous
import functools
import math

import jax
import jax.numpy as jnp
from jax.experimental import pallas as pl
from jax.experimental.pallas import tpu as pltpu

_EPS = 1e-5
_BF = jnp.bfloat16
_F32 = jnp.float32


def _row_cat(a):
    nb, H, W, C = a.shape
    zrow = jnp.zeros((nb, 1, W, C), a.dtype)
    up = jnp.concatenate([zrow, a[:, :-1]], axis=1)
    dn = jnp.concatenate([a[:, 1:], zrow], axis=1)
    return jnp.concatenate([up, a, dn], axis=3)


def _bn_affine(s, q, count, gamma, beta):
    mu = s / count
    var = q / count - mu * mu
    scale = gamma * jax.lax.rsqrt(var + _EPS)
    return scale, beta - mu * scale


def _conv_block(inp_bf, wc):
    nb, H, W, C = inp_bf.shape
    cat = _row_cat(inp_bf)
    z = jnp.dot(cat.reshape(nb * H * W, 3 * C), wc,
                preferred_element_type=_F32).reshape(nb, H, W, 256)
    zcol = jnp.zeros((nb, H, 1, C), _F32)
    t0 = jnp.concatenate([zcol, z[:, :, :-1, 0:64]], axis=2)
    t2 = jnp.concatenate([z[:, :, 1:, 192:256], zcol], axis=2)
    y = z[:, :, :, 64:128] + t0 + t2
    return z, y


def _fused_kernel(x_ref, w1_ref, ws_ref, g1_ref, be1_ref, w2_ref,
                  g2_ref, be2_ref, gs_ref, bes_ref, o_ref,
                  y1_s, ys_s, st1_s, st2_s, *, nb, count):
    p = pl.program_id(0)
    t = pl.program_id(1)
    H = W = 16
    C = 64
    S = H * W
    z64 = jnp.zeros((C, C), _F32)

    @pl.when(p == 0)
    def _conv1_phase():
        w1c = jnp.concatenate(
            [jnp.concatenate([w1_ref[ky, 0], w1_ref[ky, 1],
                              ws_ref[...] if ky == 1 else z64,
                              w1_ref[ky, 2]], axis=1) for ky in range(3)],
            axis=0).astype(_BF)
        xb = jnp.swapaxes(x_ref[...].astype(_BF), 1, 2).reshape(nb, H, W, C)
        z, y1 = _conv_block(xb, w1c)
        ys = z[:, :, :, 128:192]
        y1_s[pl.ds(t * nb, nb)] = y1.astype(_BF)
        ys_s[pl.ds(t * nb, nb)] = ys.astype(_BF)
        y1r = y1.reshape(nb * S, C)
        ysr = ys.reshape(nb * S, C)
        st = jnp.concatenate(
            [jnp.concatenate([jnp.sum(y1r, axis=0, keepdims=True),
                              jnp.sum(ysr, axis=0, keepdims=True)], axis=1),
             jnp.concatenate([jnp.sum(y1r * y1r, axis=0, keepdims=True),
                              jnp.sum(ysr * ysr, axis=0, keepdims=True)],
                             axis=1)], axis=0)

        @pl.when(t == 0)
        def _():
            st1_s[0:2, :] = st

        @pl.when(t != 0)
        def _():
            st1_s[0:2, :] = st1_s[0:2, :] + st

    @pl.when(p == 1)
    def _conv2_phase():
        w2c = jnp.concatenate(
            [jnp.concatenate([w2_ref[ky, 0], w2_ref[ky, 1], z64,
                              w2_ref[ky, 2]], axis=1) for ky in range(3)],
            axis=0).astype(_BF)
        sc, sh = _bn_affine(st1_s[0:1, 0:C], st1_s[1:2, 0:C], count,
                            g1_ref[...], be1_ref[...])
        scb = sc.astype(_BF).reshape(1, 1, 1, C)
        shb = sh.astype(_BF).reshape(1, 1, 1, C)
        y1 = y1_s[pl.ds(t * nb, nb)]
        a = jnp.maximum(y1 * scb + shb, jnp.bfloat16(0.0))
        _, y2 = _conv_block(a, w2c)
        y1_s[pl.ds(t * nb, nb)] = y2.astype(_BF)
        yr = y2.reshape(nb * S, C)
        st = jnp.concatenate(
            [jnp.sum(yr, axis=0, keepdims=True),
             jnp.sum(yr * yr, axis=0, keepdims=True)], axis=0)

        @pl.when(t == 0)
        def _():
            st2_s[0:2, :] = st

        @pl.when(t != 0)
        def _():
            st2_s[0:2, :] = st2_s[0:2, :] + st

    @pl.when(p == 2)
    def _final_phase():
        sc2, sh2 = _bn_affine(st2_s[0:1, :], st2_s[1:2, :], count,
                              g2_ref[...], be2_ref[...])
        scs, shs = _bn_affine(st1_s[0:1, C:2 * C], st1_s[1:2, C:2 * C], count,
                              gs_ref[...], bes_ref[...])
        vt = jnp.swapaxes(jnp.concatenate([sc2, sh2, scs, shs], axis=0),
                          0, 1).astype(_BF)
        y2t = jnp.swapaxes(y1_s[pl.ds(t * nb, nb)].reshape(nb, S, C), 1, 2)
        yst = jnp.swapaxes(ys_s[pl.ds(t * nb, nb)].reshape(nb, S, C), 1, 2)
        out = (y2t * vt[:, 0:1] + vt[:, 1:2]
               + yst * vt[:, 2:3] + vt[:, 3:4])
        o_ref[...] = jnp.maximum(out, jnp.bfloat16(0.0)).astype(_F32)


@jax.jit
def _forward(x, w1, g1, be1, w2, g2, be2, ws, gs, bes):
    N, Cin, H, W = x.shape
    Cout = w1.shape[-1]
    S = H * W
    x2 = x.reshape(N, Cin, S)
    count = float(N * S)

    nb = math.gcd(N, 32)
    T = N // nb
    cp = pltpu.CompilerParams(
        dimension_semantics=("arbitrary", "arbitrary"),
        vmem_limit_bytes=100 * 1024 * 1024)

    def full(a):
        return pl.BlockSpec(a.shape, lambda p, t: (0,) * a.ndim)

    o = pl.pallas_call(
        functools.partial(_fused_kernel, nb=nb, count=count),
        grid=(3, T),
        in_specs=[pl.BlockSpec((nb, Cin, S),
                               lambda p, t: (jnp.where(p == 0, t, 0), 0, 0)),
                  full(w1), full(ws), full(g1), full(be1),
                  full(w2), full(g2), full(be2), full(gs), full(bes)],
        out_specs=pl.BlockSpec((nb, Cout, S),
                               lambda p, t: (jnp.where(p == 2, t, 0), 0, 0)),
        out_shape=jax.ShapeDtypeStruct((N, Cout, S), _F32),
        scratch_shapes=[pltpu.VMEM((N, H, W, Cout), _BF),
                        pltpu.VMEM((N, H, W, Cout), _BF),
                        pltpu.VMEM((8, 2 * Cout), _F32),
                        pltpu.VMEM((8, Cout), _F32)],
        compiler_params=cp)(x2, w1, ws, g1, be1, w2, g2, be2, gs, bes)

    return o.reshape(N, Cout, H, W)


def kernel(x, w1, b1, g1, be1, w2, b2, g2, be2, ws, bs, gs, bes):
    return _forward(x, w1, g1, be1, w2, g2, be2, ws, gs, bes)
